# Initial kernel scaffold; baseline (speedup 1.0000x reference)
import functools

import jax
import jax.numpy as jnp
from jax import lax
from jax.experimental import pallas as pl
from jax.experimental.pallas import tpu as pltpu
from jax.experimental.pallas import tpu_sc as plsc

NU, NI, NBU = 8039, 32770, 4771
N = NU + NI + NBU
N_PAD = 45600
EMB = 32
E = 1_500_000
L2 = 1e-05
B = 4096
NBD = 2

NC, NS = 2, 16
NW = NC * NS
ROWS_PER_WORKER = 368
EPW = ROWS_PER_WORKER * 128
E_PAD = EPW * NW
CHUNK_ROWS = 8
NCHUNK = ROWS_PER_WORKER // CHUNK_ROWS
TROWS = N_PAD // NS

_mesh = plsc.VectorSubcoreMesh(
    core_axis_name="c", subcore_axis_name="s", num_cores=NC, num_subcores=NS)

_f32 = jnp.float32
_i32 = jnp.int32


def _scatter_body(e0_hbm, src2, dst2, w2, out_hbm,
                  accum, src_v, dst_v, w_v, rows_v, gsem, ssem):
    c = lax.axis_index("c")
    s = lax.axis_index("s")
    wid = s * NC + c

    zero = jnp.zeros((16,), _f32)

    @pl.loop(0, 1024)
    def _zero_rows(r):
        rows_v[r, pl.ds(0, 16)] = zero
        rows_v[r, pl.ds(16, 16)] = zero

    tbase = s * TROWS
    off = 0
    for ln in (1024, 1024, TROWS - 2048):
        pltpu.sync_copy(rows_v.at[pl.ds(0, ln)], accum.at[pl.ds(tbase + off, ln)])
        off += ln
    plsc.subcore_barrier()

    iota = lax.iota(_i32, 16)

    @pl.loop(0, NCHUNK)
    def _chunk(i):
        base = wid * ROWS_PER_WORKER + i * CHUNK_ROWS
        pltpu.sync_copy(src2.at[pl.ds(base, CHUNK_ROWS)], src_v)
        pltpu.sync_copy(dst2.at[pl.ds(base, CHUNK_ROWS)], dst_v)
        pltpu.sync_copy(w2.at[pl.ds(base, CHUNK_ROWS)], w_v)
        gets = [pltpu.async_copy(e0_hbm.at[src_v.at[j]],
                                 rows_v.at[pl.ds(j * 128, 128)], gsem)
                for j in range(CHUNK_ROWS)]
        for cp in gets:
            cp.wait()
        for j in range(CHUNK_ROWS):
            @pl.loop(0, 8)
            def _grp(gg, j=j):
                c0 = pl.multiple_of(gg * 16, 16)
                wv = w_v[j, pl.ds(c0, 16)]
                eidx = j * 128 + c0 + iota
                for d in range(EMB):
                    didx = jnp.full((16,), d, _i32)
                    col = plsc.load_gather(rows_v, [eidx, didx])
                    plsc.store_scatter(rows_v, [eidx, didx], col * wv)
        puts = [pltpu.async_copy(rows_v.at[pl.ds(j * 128, 128)],
                                 accum.at[dst_v.at[j]], ssem, add=True)
                for j in range(CHUNK_ROWS)]
        for cp in puts:
            cp.wait()

    plsc.subcore_barrier()
    pltpu.sync_copy(accum.at[pl.ds(tbase, TROWS)],
                    out_hbm.at[pl.ds(c * N_PAD + tbase, TROWS)])


_scatter = pl.kernel(
    _scatter_body,
    out_type=jax.ShapeDtypeStruct((NC * N_PAD, EMB), _f32),
    mesh=_mesh,
    scratch_types=[
        pltpu.VMEM_SHARED((N_PAD, EMB), _f32),
        pltpu.VMEM((CHUNK_ROWS, 128), _i32),
        pltpu.VMEM((CHUNK_ROWS, 128), _i32),
        pltpu.VMEM((CHUNK_ROWS, 128), _f32),
        pltpu.VMEM((CHUNK_ROWS * 128, EMB), _f32),
        pltpu.SemaphoreType.DMA,
        pltpu.SemaphoreType.DMA,
    ],
)

LROWS = N_PAD // NW
LCHUNK = 475


def _final_body(e0_hbm, p0_hbm, p1_hbm, users2, bundles2, ub_hbm,
                pred_out, usb_out, loss_out,
                ev, p0v, p1v, uidx, bidx, au, ubv, predv, usbv, lossv, sem):
    c = lax.axis_index("c")
    s = lax.axis_index("s")
    wid = s * NC + c
    iota = lax.iota(_i32, 16)
    zero = jnp.zeros((16,), _f32)

    acc = zero
    for k in range(3):
        rbase = wid * LROWS + k * LCHUNK
        pltpu.sync_copy(e0_hbm.at[pl.ds(rbase, LCHUNK)], ev)
        pltpu.sync_copy(p0_hbm.at[pl.ds(rbase, LCHUNK)], p0v)
        pltpu.sync_copy(p1_hbm.at[pl.ds(rbase, LCHUNK)], p1v)

        @pl.loop(0, LCHUNK, init_carry=acc)
        def _ssq(r, a):
            for h in (0, 16):
                e = ev[r, pl.ds(h, 16)]
                p = p0v[r, pl.ds(h, 16)] + p1v[r, pl.ds(h, 16)]
                v = e * 0.5 + p * (1.0 / 3.0)
                a = a + v * v
            return a
        acc = _ssq
    lossv[0, pl.ds(0, 16)] = acc
    pltpu.sync_copy(lossv, loss_out.at[pl.ds(wid, 1)])

    pltpu.sync_copy(users2.at[pl.ds(wid, 1)], uidx)
    pltpu.sync_copy(bundles2.at[pl.ds(wid * 2, 2)], bidx)
    for r in range(2):
        @pl.loop(0, 8)
        def _off(gg, r=r):
            c0 = pl.multiple_of(gg * 16, 16)
            bidx[r, pl.ds(c0, 16)] = bidx[r, pl.ds(c0, 16)] + (NU + NI)
    pltpu.sync_copy(ub_hbm, ubv)

    gets = []
    for tbl, dstv in ((e0_hbm, ev), (p0_hbm, p0v), (p1_hbm, p1v)):
        gets.append(pltpu.async_copy(tbl.at[uidx.at[0]], dstv.at[pl.ds(0, 128)], sem))
        gets.append(pltpu.async_copy(tbl.at[bidx.at[0]], dstv.at[pl.ds(128, 128)], sem))
        gets.append(pltpu.async_copy(tbl.at[bidx.at[1]], dstv.at[pl.ds(256, 128)], sem))
    for cp in gets:
        cp.wait()

    @pl.loop(0, 384)
    def _cmb(r):
        for h in (0, 16):
            v = ev[r, pl.ds(h, 16)] * 0.5 + \
                (p0v[r, pl.ds(h, 16)] + p1v[r, pl.ds(h, 16)]) * (1.0 / 3.0)
            au[r, pl.ds(h, 16)] = v

    @pl.loop(0, 16)
    def _pred(g):
        q0 = pl.multiple_of(g * 16, 16)
        qv = q0 + iota
        urow = lax.div(qv, 2)
        brow = 128 + qv
        a = zero
        for d in range(EMB):
            didx = jnp.full((16,), d, _i32)
            ucol = plsc.load_gather(au, [urow, didx])
            bcol = plsc.load_gather(au, [brow, didx])
            a = a + ucol * bcol
        predv[pl.ds(q0, 16)] = a
    pltpu.sync_copy(predv, pred_out.at[pl.ds(wid * 256, 256)])

    @pl.loop(0, 8)
    def _usb(g):
        r0 = pl.multiple_of(g * 16, 16)
        rows = r0 + iota
        a = zero
        for d in range(EMB):
            didx = jnp.full((16,), d, _i32)
            ucol = plsc.load_gather(au, [rows, didx])
            wb = plsc.load_gather(ubv, [didx])
            a = a + ucol * wb
        usbv[pl.ds(r0, 16)] = a
    pltpu.sync_copy(usbv, usb_out.at[pl.ds(wid * 128, 128)])


_final = pl.kernel(
    _final_body,
    out_type=(
        jax.ShapeDtypeStruct((B * NBD,), _f32),
        jax.ShapeDtypeStruct((B,), _f32),
        jax.ShapeDtypeStruct((NW, 16), _f32),
    ),
    mesh=_mesh,
    scratch_types=[
        pltpu.VMEM((LCHUNK, EMB), _f32),
        pltpu.VMEM((LCHUNK, EMB), _f32),
        pltpu.VMEM((LCHUNK, EMB), _f32),
        pltpu.VMEM((1, 128), _i32),
        pltpu.VMEM((2, 128), _i32),
        pltpu.VMEM((384, EMB), _f32),
        pltpu.VMEM((EMB,), _f32),
        pltpu.VMEM((256,), _f32),
        pltpu.VMEM((128,), _f32),
        pltpu.VMEM((1, 16), _f32),
        pltpu.SemaphoreType.DMA,
    ],
)


def kernel(users, bundles, src, dst, w, users_feature, items_feature,
           bundles_feature, user_bound):
    e0 = jnp.concatenate(
        [users_feature, items_feature, bundles_feature,
         jnp.zeros((N_PAD - N, EMB), _f32)], axis=0)
    pad = E_PAD - E
    src2 = jnp.concatenate([src, jnp.zeros((pad,), _i32)]).reshape(-1, 128)
    dst2 = jnp.concatenate([dst, jnp.zeros((pad,), _i32)]).reshape(-1, 128)
    w2 = jnp.concatenate([w, jnp.zeros((pad,), _f32)]).reshape(-1, 128)

    parts = _scatter(e0, src2, dst2, w2)
    p0 = parts[:N_PAD]
    p1 = parts[N_PAD:]

    users2 = users.reshape(NW, 128)
    bundles2 = bundles.reshape(NW * 2, 128)
    pred_f, usb_f, loss_p = _final(e0, p0, p1, users2, bundles2,
                                   user_bound.reshape(EMB))
    pred = pred_f.reshape(B, NBD)
    usb = usb_f.reshape(B, 1, 1)
    loss = _f32(L2) * jnp.sum(loss_p)
    return (pred, usb, loss)


# speedup vs baseline: 16.7796x; 16.7796x over previous
import functools

import jax
import jax.numpy as jnp
from jax import lax
from jax.experimental import pallas as pl
from jax.experimental.pallas import tpu as pltpu
from jax.experimental.pallas import tpu_sc as plsc

NU, NI, NBU = 8039, 32770, 4771
N = NU + NI + NBU
N_PAD = 45824
EMB = 32
E = 1_500_000
L2 = 1e-05
B = 4096
NBD = 2

NC, NS = 2, 16
NW = NC * NS
ROWS_PER_WORKER = 368
EPW = ROWS_PER_WORKER * 128
E_PAD = EPW * NW
CHUNK_ROWS = 8
NCHUNK = ROWS_PER_WORKER // CHUNK_ROWS
TROWS = N_PAD // NS

_mesh = plsc.VectorSubcoreMesh(
    core_axis_name="c", subcore_axis_name="s", num_cores=NC, num_subcores=NS)

_f32 = jnp.float32
_i32 = jnp.int32


def _scatter_body(e0_hbm, src2, dst2, w2, out_hbm,
                  accum, src_v, dst_v, w_v, rows_v, gsem, ssem):
    c = lax.axis_index("c")
    s = lax.axis_index("s")
    wid = s * NC + c

    zero = jnp.zeros((16,), _f32)

    @pl.loop(0, 1024)
    def _zero_rows(r):
        rows_v[r, pl.ds(0, 16)] = zero
        rows_v[r, pl.ds(16, 16)] = zero

    tbase = s * TROWS
    off = 0
    for ln in (1024, 1024, TROWS - 2048):
        pltpu.sync_copy(rows_v.at[pl.ds(0, ln)], accum.at[pl.ds(tbase + off, ln)])
        off += ln
    plsc.subcore_barrier()

    iota = lax.iota(_i32, 16)

    @pl.loop(0, NCHUNK)
    def _chunk(i):
        base = wid * ROWS_PER_WORKER + i * CHUNK_ROWS
        pltpu.sync_copy(src2.at[pl.ds(base, CHUNK_ROWS)], src_v)
        pltpu.sync_copy(dst2.at[pl.ds(base, CHUNK_ROWS)], dst_v)
        pltpu.sync_copy(w2.at[pl.ds(base, CHUNK_ROWS)], w_v)
        gets = [pltpu.async_copy(e0_hbm.at[src_v.at[j]],
                                 rows_v.at[pl.ds(j * 128, 128)], gsem)
                for j in range(CHUNK_ROWS)]
        for cp in gets:
            cp.wait()
        for j in range(CHUNK_ROWS):
            @pl.loop(0, 8)
            def _grp(gg, j=j):
                c0 = pl.multiple_of(gg * 16, 16)
                wv = w_v[j, pl.ds(c0, 16)]
                e0 = j * 128 + c0
                for l in range(16):
                    e = e0 + l
                    wb = jnp.full((16,), wv[l], _f32)
                    rows_v[e, pl.ds(0, 16)] = rows_v[e, pl.ds(0, 16)] * wb
                    rows_v[e, pl.ds(16, 16)] = rows_v[e, pl.ds(16, 16)] * wb
        puts = [pltpu.async_copy(rows_v.at[pl.ds(j * 128, 128)],
                                 accum.at[dst_v.at[j]], ssem, add=True)
                for j in range(CHUNK_ROWS)]
        for cp in puts:
            cp.wait()

    plsc.subcore_barrier()
    pltpu.sync_copy(accum.at[pl.ds(tbase, TROWS)],
                    out_hbm.at[pl.ds(c * N_PAD + tbase, TROWS)])


_scatter = pl.kernel(
    _scatter_body,
    out_type=jax.ShapeDtypeStruct((NC * N_PAD, EMB), _f32),
    mesh=_mesh,
    compiler_params=pltpu.CompilerParams(use_tc_tiling_on_sc=False),
    scratch_types=[
        pltpu.VMEM_SHARED((N_PAD, EMB), _f32),
        pltpu.VMEM((CHUNK_ROWS, 128), _i32),
        pltpu.VMEM((CHUNK_ROWS, 128), _i32),
        pltpu.VMEM((CHUNK_ROWS, 128), _f32),
        pltpu.VMEM((CHUNK_ROWS * 128, EMB), _f32),
        pltpu.SemaphoreType.DMA,
        pltpu.SemaphoreType.DMA,
    ],
)

GROWS = 384


def _gather_body(e0_hbm, p0_hbm, p1_hbm, users2, bundles2,
                 u_out, bz_out,
                 ev, p0v, p1v, uidx, bidx, au, sem):
    c = lax.axis_index("c")
    s = lax.axis_index("s")
    wid = s * NC + c

    pltpu.sync_copy(users2, uidx)
    pltpu.sync_copy(bundles2, bidx)
    for r in range(2):
        @pl.loop(0, 8)
        def _off(gg, r=r):
            c0 = pl.multiple_of(gg * 16, 16)
            r2 = wid * 2 + r
            bidx[r2, pl.ds(c0, 16)] = bidx[r2, pl.ds(c0, 16)] + (NU + NI)

    gets = []
    for tbl, dstv in ((e0_hbm, ev), (p0_hbm, p0v), (p1_hbm, p1v)):
        gets.append(pltpu.async_copy(tbl.at[uidx.at[wid]], dstv.at[pl.ds(0, 128)], sem))
        gets.append(pltpu.async_copy(tbl.at[bidx.at[wid * 2]], dstv.at[pl.ds(128, 128)], sem))
        gets.append(pltpu.async_copy(tbl.at[bidx.at[wid * 2 + 1]], dstv.at[pl.ds(256, 128)], sem))
    for cp in gets:
        cp.wait()

    @pl.loop(0, GROWS)
    def _cmb(r):
        for h in (0, 16):
            v = ev[r, pl.ds(h, 16)] * 0.5 + \
                (p0v[r, pl.ds(h, 16)] + p1v[r, pl.ds(h, 16)]) * (1.0 / 3.0)
            au[r, pl.ds(h, 16)] = v

    pltpu.sync_copy(au.at[pl.ds(0, 128)], u_out.at[pl.ds(wid * 128, 128)])
    pltpu.sync_copy(au.at[pl.ds(128, 256)], bz_out.at[pl.ds(wid * 256, 256)])


_gather = pl.kernel(
    _gather_body,
    out_type=(
        jax.ShapeDtypeStruct((B, EMB), _f32),
        jax.ShapeDtypeStruct((B * NBD, EMB), _f32),
    ),
    mesh=_mesh,
    compiler_params=pltpu.CompilerParams(use_tc_tiling_on_sc=False),
    scratch_types=[
        pltpu.VMEM((GROWS, EMB), _f32),
        pltpu.VMEM((GROWS, EMB), _f32),
        pltpu.VMEM((GROWS, EMB), _f32),
        pltpu.VMEM((NW, 128), _i32),
        pltpu.VMEM((NW * 2, 128), _i32),
        pltpu.VMEM((GROWS, EMB), _f32),
        pltpu.SemaphoreType.DMA,
    ],
)


def _head_body(e0_ref, p0_ref, p1_ref, u_ref, bz_ref, ub_ref,
               pred_ref, usb_ref, loss_ref):
    a = e0_ref[...] * 0.5 + (p0_ref[...] + p1_ref[...]) * (1.0 / 3.0)
    loss_ref[...] = jnp.full((8, 128), jnp.sum(a * a), _f32)
    u = u_ref[...]
    bz = bz_ref[...]
    b0 = bz[:, :EMB]
    b1 = bz[:, EMB:]
    pred0 = jnp.sum(u * b0, axis=1, keepdims=True)
    pred1 = jnp.sum(u * b1, axis=1, keepdims=True)
    pred_ref[...] = jnp.concatenate([pred0, pred1], axis=1)
    usb_ref[...] = jax.lax.dot_general(
        u, ub_ref[...], (((1,), (0,)), ((), ())),
        preferred_element_type=_f32)


def _head(e0r, p0r, p1r, u, bz, ub):
    return pl.pallas_call(
        _head_body,
        out_shape=(
            jax.ShapeDtypeStruct((B, NBD), _f32),
            jax.ShapeDtypeStruct((B, 1), _f32),
            jax.ShapeDtypeStruct((8, 128), _f32),
        ),
    )(e0r, p0r, p1r, u, bz, ub)


def kernel(users, bundles, src, dst, w, users_feature, items_feature,
           bundles_feature, user_bound):
    e0 = jnp.concatenate(
        [users_feature, items_feature, bundles_feature,
         jnp.zeros((N_PAD - N, EMB), _f32)], axis=0)
    pad = E_PAD - E
    src2 = jnp.concatenate([src, jnp.zeros((pad,), _i32)]).reshape(-1, 128)
    dst2 = jnp.concatenate([dst, jnp.zeros((pad,), _i32)]).reshape(-1, 128)
    w2 = jnp.concatenate([w, jnp.zeros((pad,), _f32)]).reshape(-1, 128)

    parts = _scatter(e0, src2, dst2, w2)
    p0 = parts[:N_PAD]
    p1 = parts[N_PAD:]

    users2 = users.reshape(NW, 128)
    bundles2 = bundles.reshape(NW * 2, 128)
    u, bz = _gather(e0, p0, p1, users2, bundles2)

    pred, usb, loss8 = _head(
        e0.reshape(-1, 128), p0.reshape(-1, 128), p1.reshape(-1, 128),
        u, bz.reshape(B, NBD * EMB), user_bound)
    loss = _f32(L2) * loss8[0, 0]
    return (pred, usb.reshape(B, 1, 1), loss)
